# Initial kernel scaffold; baseline (speedup 1.0000x reference)
import jax
import jax.numpy as jnp
from jax.experimental import pallas as pl


def kernel(x, row, col, batch, W_root, b_root, W_rel):
    raise NotImplementedError("write your pallas kernel here")


# speedup vs baseline: 7.7474x; 7.7474x over previous
import functools

import jax
import jax.numpy as jnp
from jax import lax
from jax.experimental import pallas as pl
from jax.experimental.pallas import tpu as pltpu
from jax.experimental.pallas import tpu_sc as plsc

N = 10000
E = 320000
D = 128

NC = 2
NS = 16
NW = NC * NS

E_PER_W = E // NW
K = 80
NCHUNK = E_PER_W // K
NP = 10240
ROWS_PER_S = NP // NS


def _sc_scatter_build():
    mesh = plsc.VectorSubcoreMesh(core_axis_name="c", subcore_axis_name="s")

    @functools.partial(
        pl.kernel,
        out_type=jax.ShapeDtypeStruct((NC, NP, D), jnp.float32),
        mesh=mesh,
        scratch_types=[
            pltpu.VMEM((NCHUNK, K), jnp.int32),
            pltpu.VMEM((NCHUNK, K), jnp.int32),
            pltpu.VMEM((K, D), jnp.float32),
            pltpu.VMEM_SHARED((NP, D), jnp.float32),
            pltpu.SemaphoreType.DMA,
        ],
    )
    def sc_scatter(x_hbm, row_hbm, col_hbm, zeros_hbm, out_hbm,
                   row_v, col_v, rows_v, acc, sem):
        c = lax.axis_index("c")
        s = lax.axis_index("s")
        wid = s * NC + c

        pltpu.sync_copy(zeros_hbm, acc.at[pl.ds(s * ROWS_PER_S, ROWS_PER_S)])

        pltpu.sync_copy(row_hbm.at[wid], row_v)
        pltpu.sync_copy(col_hbm.at[wid], col_v)

        plsc.subcore_barrier()

        def body(j, carry):
            pltpu.async_copy(x_hbm.at[row_v.at[j]], rows_v, sem).wait()
            pltpu.sync_copy(rows_v, acc.at[col_v.at[j]], add=True)
            return carry

        lax.fori_loop(0, NCHUNK, body, 0)

        plsc.subcore_barrier()

        pltpu.sync_copy(acc.at[pl.ds(s * ROWS_PER_S, ROWS_PER_S)],
                        out_hbm.at[c, pl.ds(s * ROWS_PER_S, ROWS_PER_S)])

    return sc_scatter


_sc_scatter = _sc_scatter_build()


BN = 1000


def _tc_finish_body(part_ref, x_ref, wroot_ref, b_ref, wrel_ref, out_ref):
    agg = part_ref[0] + part_ref[1]
    dn = (((1,), (1,)), ((), ()))
    rel = lax.dot_general(agg, wrel_ref[...], dn,
                          preferred_element_type=jnp.float32)
    root = lax.dot_general(x_ref[...], wroot_ref[...], dn,
                           preferred_element_type=jnp.float32)
    out_ref[...] = jnp.maximum(rel + root + b_ref[...], 0.0)


def _tc_finish(part, x, W_root, b_root, W_rel):
    grid = (N // BN,)
    return pl.pallas_call(
        _tc_finish_body,
        grid=grid,
        in_specs=[
            pl.BlockSpec((NC, BN, D), lambda i: (0, i, 0)),
            pl.BlockSpec((BN, D), lambda i: (i, 0)),
            pl.BlockSpec((D, D), lambda i: (0, 0)),
            pl.BlockSpec((1, D), lambda i: (0, 0)),
            pl.BlockSpec((D, D), lambda i: (0, 0)),
        ],
        out_specs=pl.BlockSpec((BN, D), lambda i: (i, 0)),
        out_shape=jax.ShapeDtypeStruct((N, D), jnp.float32),
    )(part, x, W_root, b_root.reshape(1, D), W_rel)


def kernel(x, row, col, batch, W_root, b_root, W_rel):
    row32 = row.astype(jnp.int32).reshape(NW, NCHUNK, K)
    col32 = col.astype(jnp.int32).reshape(NW, NCHUNK, K)
    zeros = jnp.zeros((ROWS_PER_S, D), jnp.float32)
    part = _sc_scatter(x, row32, col32, zeros)
    return _tc_finish(part, x, W_root, b_root, W_rel)
